# initial kernel scaffold (unmeasured)
import jax
import jax.numpy as jnp
from jax import lax
from jax.experimental import pallas as pl
from jax.experimental.pallas import tpu as pltpu

N_DEV = 4


def kernel(x, dy):
    partial = lax.dot_general(
        x.astype(jnp.bfloat16),
        dy.astype(jnp.bfloat16),
        dimension_numbers=(((0,), (0,)), ((), ())),
        preferred_element_type=jnp.float32,
    )

    m, n = partial.shape
    m_out = m // N_DEV

    def body(partial_ref, out_ref, comm_ref, local_ref,
             send_sems, recv_sems, copy_sem, out_sem, credit_sem):
        mx = lax.axis_index("x")
        my = lax.axis_index("y")
        mz = lax.axis_index("z")
        left = lax.rem(mz + N_DEV - 1, N_DEV)
        right = lax.rem(mz + 1, N_DEV)

        barrier = pltpu.get_barrier_semaphore()
        for nbr in (left, right):
            pl.semaphore_signal(barrier, inc=1, device_id=(mx, my, nbr),
                                device_id_type=pl.DeviceIdType.MESH)
        pl.semaphore_wait(barrier, 2)

        def chunk(c):
            return partial_ref.at[pl.ds(c * m_out, m_out), :]

        c0 = lax.rem(mz + N_DEV - 1, N_DEV)
        cp = pltpu.make_async_copy(chunk(c0), comm_ref.at[0], copy_sem)
        cp.start()
        cp.wait()

        for h in range(N_DEV - 1):
            s_slot = h % 2
            r_slot = (h + 1) % 2
            if h > 0:
                pl.semaphore_wait(credit_sem, 1)
            rdma = pltpu.make_async_remote_copy(
                src_ref=comm_ref.at[s_slot],
                dst_ref=comm_ref.at[r_slot],
                send_sem=send_sems.at[h],
                recv_sem=recv_sems.at[h],
                device_id=(mx, my, right),
                device_id_type=pl.DeviceIdType.MESH,
            )
            rdma.start()
            c_in = lax.rem(mz + 2 * N_DEV - 2 - h, N_DEV)
            cp = pltpu.make_async_copy(chunk(c_in), local_ref, copy_sem)
            cp.start()
            rdma.wait()
            if h < N_DEV - 2:
                pl.semaphore_signal(credit_sem, inc=1,
                                    device_id=(mx, my, left),
                                    device_id_type=pl.DeviceIdType.MESH)
            cp.wait()
            comm_ref[r_slot] = comm_ref[r_slot] + local_ref[...]

        out_cp = pltpu.make_async_copy(comm_ref.at[(N_DEV - 1) % 2], out_ref,
                                       out_sem)
        out_cp.start()
        out_cp.wait()

    return pl.pallas_call(
        body,
        out_shape=jax.ShapeDtypeStruct((m_out, n), jnp.float32),
        in_specs=[pl.BlockSpec(memory_space=pltpu.ANY)],
        out_specs=pl.BlockSpec(memory_space=pltpu.ANY),
        scratch_shapes=[
            pltpu.VMEM((2, m_out, n), jnp.float32),
            pltpu.VMEM((m_out, n), jnp.float32),
            pltpu.SemaphoreType.DMA((N_DEV - 1,)),
            pltpu.SemaphoreType.DMA((N_DEV - 1,)),
            pltpu.SemaphoreType.DMA,
            pltpu.SemaphoreType.DMA,
            pltpu.SemaphoreType.REGULAR,
        ],
        compiler_params=pltpu.CompilerParams(collective_id=0),
    )(partial)


# baseline (device time: 693601 ns/iter reference)
import jax
import jax.numpy as jnp
from jax import lax
from jax.experimental import pallas as pl
from jax.experimental.pallas import tpu as pltpu

N_DEV = 4


def kernel(x, dy):
    partial = lax.dot_general(
        x.astype(jnp.bfloat16),
        dy.astype(jnp.bfloat16),
        dimension_numbers=(((0,), (0,)), ((), ())),
        preferred_element_type=jnp.float32,
    )

    m, n = partial.shape
    m_out = m // N_DEV

    def body(partial_ref, out_ref, comm_ref, local_ref,
             send_sems, recv_sems, copy_sem, out_sem, credit_sem):
        mx = lax.axis_index("x")
        my = lax.axis_index("y")
        mz = lax.axis_index("z")
        left = lax.rem(mz + N_DEV - 1, N_DEV)
        right = lax.rem(mz + 1, N_DEV)

        barrier = pltpu.get_barrier_semaphore()
        for nbr in (left, right):
            pl.semaphore_signal(barrier, inc=1, device_id=(mx, my, nbr),
                                device_id_type=pl.DeviceIdType.MESH)
        pl.semaphore_wait(barrier, 2)

        def chunk(c):
            return partial_ref.at[pl.ds(c * m_out, m_out), :]

        c0 = lax.rem(mz + N_DEV - 1, N_DEV)
        cp = pltpu.make_async_copy(chunk(c0), comm_ref.at[0], copy_sem)
        cp.start()
        cp.wait()

        for h in range(N_DEV - 1):
            s_slot = h % 2
            r_slot = (h + 1) % 2
            if h > 0:
                pl.semaphore_wait(credit_sem, 1)
            rdma = pltpu.make_async_remote_copy(
                src_ref=comm_ref.at[s_slot],
                dst_ref=comm_ref.at[r_slot],
                send_sem=send_sems.at[h],
                recv_sem=recv_sems.at[h],
                device_id=(mx, my, right),
                device_id_type=pl.DeviceIdType.MESH,
            )
            rdma.start()
            c_in = lax.rem(mz + 2 * N_DEV - 2 - h, N_DEV)
            cp = pltpu.make_async_copy(chunk(c_in), local_ref, copy_sem)
            cp.start()
            rdma.wait()
            if h < N_DEV - 2:
                pl.semaphore_signal(credit_sem, inc=1,
                                    device_id=(mx, my, left),
                                    device_id_type=pl.DeviceIdType.MESH)
            cp.wait()
            comm_ref[r_slot] = comm_ref[r_slot] + local_ref[...]

        out_cp = pltpu.make_async_copy(comm_ref.at[(N_DEV - 1) % 2], out_ref,
                                       out_sem)
        out_cp.start()
        out_cp.wait()

    return pl.pallas_call(
        body,
        out_shape=jax.ShapeDtypeStruct((m_out, n), jnp.float32),
        in_specs=[pl.BlockSpec(memory_space=pltpu.MemorySpace.HBM)],
        out_specs=pl.BlockSpec(memory_space=pltpu.MemorySpace.HBM),
        scratch_shapes=[
            pltpu.VMEM((2, m_out, n), jnp.float32),
            pltpu.VMEM((m_out, n), jnp.float32),
            pltpu.SemaphoreType.DMA((N_DEV - 1,)),
            pltpu.SemaphoreType.DMA((N_DEV - 1,)),
            pltpu.SemaphoreType.DMA,
            pltpu.SemaphoreType.DMA,
            pltpu.SemaphoreType.REGULAR,
        ],
        compiler_params=pltpu.CompilerParams(
            collective_id=0, vmem_limit_bytes=60 * 1024 * 1024
        ),
    )(partial)


# device time: 296240 ns/iter; 2.3413x vs baseline; 2.3413x over previous
import jax
import jax.numpy as jnp
from jax import lax
from jax.experimental import pallas as pl
from jax.experimental.pallas import tpu as pltpu

NZ = 4
NR = 8
MESH = pl.DeviceIdType.MESH


def _ring_coords(q):
    qx = jnp.where(q < 4, 0, 1)
    qy = jnp.where(q < 4, q, 7 - q)
    return qx, qy


def kernel(x, dy):
    k_loc, m = x.shape
    n_tot = dy.shape[1]
    n_grp = n_tot // NR
    m_out = m // NZ

    mx = lax.axis_index("x")
    my = lax.axis_index("y")
    p = jnp.where(mx == 0, my, NR - 1 - my)

    dy_g = lax.dynamic_slice(dy, (0, p * n_grp), (k_loc, n_grp))
    partial = lax.dot_general(
        x.astype(jnp.bfloat16),
        dy_g.astype(jnp.bfloat16),
        dimension_numbers=(((0,), (0,)), ((), ())),
        preferred_element_type=jnp.float32,
    )

    def body(partial_ref, out_ref, comm_ref,
             send1, recv1, send2, recv2, credit1, credit2):
        mx = lax.axis_index("x")
        my = lax.axis_index("y")
        mz = lax.axis_index("z")
        p = jnp.where(mx == 0, my, NR - 1 - my)
        zl = lax.rem(mz + NZ - 1, NZ)
        zr = lax.rem(mz + 1, NZ)
        nxt = _ring_coords(lax.rem(p + 1, NR))
        prv = _ring_coords(lax.rem(p + NR - 1, NR))

        barrier = pltpu.get_barrier_semaphore()
        for nbr in (zl, zr):
            pl.semaphore_signal(barrier, inc=1, device_id=(mx, my, nbr),
                                device_id_type=MESH)
        pl.semaphore_wait(barrier, 2)

        def chunk(c):
            return partial_ref[pl.ds(c * m_out, m_out), :]

        comm_ref[0] = chunk(lax.rem(mz + NZ - 1, NZ))
        for h in range(NZ - 1):
            s_slot, r_slot = h % 2, (h + 1) % 2
            if h > 0:
                pl.semaphore_wait(credit1, 1)
            rdma = pltpu.make_async_remote_copy(
                src_ref=comm_ref.at[s_slot],
                dst_ref=comm_ref.at[r_slot],
                send_sem=send1.at[h],
                recv_sem=recv1.at[h],
                device_id=(mx, my, zr),
                device_id_type=MESH,
            )
            rdma.start()
            rdma.wait()
            if h < NZ - 2:
                pl.semaphore_signal(credit1, inc=1, device_id=(mx, my, zl),
                                    device_id_type=MESH)
            comm_ref[r_slot] = comm_ref[r_slot] + chunk(lax.rem(mz + 2 * NZ - 2 - h, NZ))

        out_ref[:, pl.ds(p * n_grp, n_grp)] = comm_ref[1]

        pl.semaphore_signal(credit2, inc=1, device_id=(*prv, mz),
                            device_id_type=MESH)
        for k in range(NR - 1):
            s_slot, r_slot = (1 + k) % 2, k % 2
            pl.semaphore_wait(credit2, 1)
            rdma = pltpu.make_async_remote_copy(
                src_ref=comm_ref.at[s_slot],
                dst_ref=comm_ref.at[r_slot],
                send_sem=send2.at[k],
                recv_sem=recv2.at[k],
                device_id=(*nxt, mz),
                device_id_type=MESH,
            )
            rdma.start()
            rdma.wait()
            if k < NR - 2:
                pl.semaphore_signal(credit2, inc=1, device_id=(*prv, mz),
                                    device_id_type=MESH)
            origin = lax.rem(p + NR - 1 - k, NR)
            out_ref[:, pl.ds(origin * n_grp, n_grp)] = comm_ref[r_slot]

    return pl.pallas_call(
        body,
        out_shape=jax.ShapeDtypeStruct((m_out, n_tot), jnp.float32),
        in_specs=[pl.BlockSpec(memory_space=pltpu.MemorySpace.VMEM)],
        out_specs=pl.BlockSpec(memory_space=pltpu.MemorySpace.VMEM),
        scratch_shapes=[
            pltpu.VMEM((2, m_out, n_grp), jnp.float32),
            pltpu.SemaphoreType.DMA((NZ - 1,)),
            pltpu.SemaphoreType.DMA((NZ - 1,)),
            pltpu.SemaphoreType.DMA((NR - 1,)),
            pltpu.SemaphoreType.DMA((NR - 1,)),
            pltpu.SemaphoreType.REGULAR,
            pltpu.SemaphoreType.REGULAR,
        ],
        compiler_params=pltpu.CompilerParams(
            collective_id=0, vmem_limit_bytes=48 * 1024 * 1024
        ),
    )(partial)


# device time: 220628 ns/iter; 3.1438x vs baseline; 1.3427x over previous
import jax
import jax.numpy as jnp
from jax import lax
from jax.experimental import pallas as pl
from jax.experimental.pallas import tpu as pltpu

NZ = 4
NR = 8
MESH = pl.DeviceIdType.MESH


def _ring_coords(q):
    qx = jnp.where(q < 4, 0, 1)
    qy = jnp.where(q < 4, q, 7 - q)
    return qx, qy


def kernel(x, dy):
    k_loc, m = x.shape
    n_tot = dy.shape[1]
    n_grp = n_tot // NR
    m_out = m // NZ

    mx = lax.axis_index("x")
    my = lax.axis_index("y")
    p = jnp.where(mx == 0, my, NR - 1 - my)

    dy_g = lax.dynamic_slice(dy, (0, p * n_grp), (k_loc, n_grp))
    partial = lax.dot_general(
        x.astype(jnp.bfloat16),
        dy_g.astype(jnp.bfloat16),
        dimension_numbers=(((0,), (0,)), ((), ())),
        preferred_element_type=jnp.float32,
    )

    n_half = n_grp // 2

    def body(partial_ref, out_ref, comm_ref, commA, commB,
             send1, recv1, sendA, recvA, sendB, recvB,
             credit1, creditA, creditB):
        mx = lax.axis_index("x")
        my = lax.axis_index("y")
        mz = lax.axis_index("z")
        p = jnp.where(mx == 0, my, NR - 1 - my)
        zl = lax.rem(mz + NZ - 1, NZ)
        zr = lax.rem(mz + 1, NZ)
        nxt = _ring_coords(lax.rem(p + 1, NR))
        prv = _ring_coords(lax.rem(p + NR - 1, NR))

        barrier = pltpu.get_barrier_semaphore()
        for nbr in (zl, zr):
            pl.semaphore_signal(barrier, inc=1, device_id=(mx, my, nbr),
                                device_id_type=MESH)
        pl.semaphore_wait(barrier, 2)

        def chunk(c):
            return partial_ref[pl.ds(c * m_out, m_out), :]

        comm_ref[0] = chunk(lax.rem(mz + NZ - 1, NZ))
        for h in range(NZ - 1):
            s_slot, r_slot = h % 2, (h + 1) % 2
            if h > 0:
                pl.semaphore_wait(credit1, 1)
            rdma = pltpu.make_async_remote_copy(
                src_ref=comm_ref.at[s_slot],
                dst_ref=comm_ref.at[r_slot],
                send_sem=send1.at[h],
                recv_sem=recv1.at[h],
                device_id=(mx, my, zr),
                device_id_type=MESH,
            )
            rdma.start()
            rdma.wait()
            if h < NZ - 2:
                pl.semaphore_signal(credit1, inc=1, device_id=(mx, my, zl),
                                    device_id_type=MESH)
            comm_ref[r_slot] = comm_ref[r_slot] + chunk(lax.rem(mz + 2 * NZ - 2 - h, NZ))

        commA[1] = comm_ref[1, :, :n_half]
        commB[1] = comm_ref[1, :, n_half:]

        pl.semaphore_signal(creditA, inc=1, device_id=(*prv, mz),
                            device_id_type=MESH)
        pl.semaphore_signal(creditB, inc=1, device_id=(*nxt, mz),
                            device_id_type=MESH)
        for k in range(NR - 1):
            s_slot, r_slot = (1 + k) % 2, k % 2
            pl.semaphore_wait(creditA, 1)
            pl.semaphore_wait(creditB, 1)
            rdma_a = pltpu.make_async_remote_copy(
                src_ref=commA.at[s_slot],
                dst_ref=commA.at[r_slot],
                send_sem=sendA.at[k],
                recv_sem=recvA.at[k],
                device_id=(*nxt, mz),
                device_id_type=MESH,
            )
            rdma_b = pltpu.make_async_remote_copy(
                src_ref=commB.at[s_slot],
                dst_ref=commB.at[r_slot],
                send_sem=sendB.at[k],
                recv_sem=recvB.at[k],
                device_id=(*prv, mz),
                device_id_type=MESH,
            )
            rdma_a.start()
            rdma_b.start()
            if k == 0:
                out_ref[:, pl.ds(p * n_grp, n_grp)] = comm_ref[1]
            else:
                oa = lax.rem(p + NR - k, NR)
                ob = lax.rem(p + k, NR)
                out_ref[:, pl.ds(oa * n_grp, n_half)] = commA[s_slot]
                out_ref[:, pl.ds(ob * n_grp + n_half, n_half)] = commB[s_slot]
            rdma_a.wait()
            rdma_b.wait()
            if k < NR - 2:
                pl.semaphore_signal(creditA, inc=1, device_id=(*prv, mz),
                                    device_id_type=MESH)
                pl.semaphore_signal(creditB, inc=1, device_id=(*nxt, mz),
                                    device_id_type=MESH)
        oa = lax.rem(p + 1, NR)
        ob = lax.rem(p + NR - 1, NR)
        out_ref[:, pl.ds(oa * n_grp, n_half)] = commA[0]
        out_ref[:, pl.ds(ob * n_grp + n_half, n_half)] = commB[0]

    return pl.pallas_call(
        body,
        out_shape=jax.ShapeDtypeStruct((m_out, n_tot), jnp.float32),
        in_specs=[pl.BlockSpec(memory_space=pltpu.MemorySpace.VMEM)],
        out_specs=pl.BlockSpec(memory_space=pltpu.MemorySpace.VMEM),
        scratch_shapes=[
            pltpu.VMEM((2, m_out, n_grp), jnp.float32),
            pltpu.VMEM((2, m_out, n_grp // 2), jnp.float32),
            pltpu.VMEM((2, m_out, n_grp // 2), jnp.float32),
            pltpu.SemaphoreType.DMA((NZ - 1,)),
            pltpu.SemaphoreType.DMA((NZ - 1,)),
            pltpu.SemaphoreType.DMA((NR - 1,)),
            pltpu.SemaphoreType.DMA((NR - 1,)),
            pltpu.SemaphoreType.DMA((NR - 1,)),
            pltpu.SemaphoreType.DMA((NR - 1,)),
            pltpu.SemaphoreType.REGULAR,
            pltpu.SemaphoreType.REGULAR,
            pltpu.SemaphoreType.REGULAR,
        ],
        compiler_params=pltpu.CompilerParams(
            collective_id=0, vmem_limit_bytes=48 * 1024 * 1024
        ),
    )(partial)


# device time: 146921 ns/iter; 4.7209x vs baseline; 1.5017x over previous
import jax
import jax.numpy as jnp
from jax import lax
from jax.experimental import pallas as pl
from jax.experimental.pallas import tpu as pltpu

NZ = 4
NR = 8
MESH = pl.DeviceIdType.MESH


def _ring_coords(q):
    qx = jnp.where(q < 4, 0, 1)
    qy = jnp.where(q < 4, q, 7 - q)
    return qx, qy


def kernel(x, dy):
    k_loc, m = x.shape
    n_tot = dy.shape[1]
    n_grp = n_tot // NR
    m_out = m // NZ

    mx = lax.axis_index("x")
    my = lax.axis_index("y")
    p = jnp.where(mx == 0, my, NR - 1 - my)

    dy_g = lax.dynamic_slice(dy, (0, p * n_grp), (k_loc, n_grp))
    partial = lax.dot_general(
        x.astype(jnp.bfloat16),
        dy_g.astype(jnp.bfloat16),
        dimension_numbers=(((0,), (0,)), ((), ())),
        preferred_element_type=jnp.float32,
    )

    n_half = n_grp // 2

    def body(partial_ref, out_ref, comm_ref, commA, commB,
             send1, recv1, sendA, recvA, sendB, recvB,
             credit1, creditA, creditB):
        mx = lax.axis_index("x")
        my = lax.axis_index("y")
        mz = lax.axis_index("z")
        p = jnp.where(mx == 0, my, NR - 1 - my)
        zl = lax.rem(mz + NZ - 1, NZ)
        zr = lax.rem(mz + 1, NZ)
        nxt = _ring_coords(lax.rem(p + 1, NR))
        prv = _ring_coords(lax.rem(p + NR - 1, NR))

        barrier = pltpu.get_barrier_semaphore()
        for nbr in (zl, zr):
            pl.semaphore_signal(barrier, inc=1, device_id=(mx, my, nbr),
                                device_id_type=MESH)
        pl.semaphore_wait(barrier, 2)

        def chunk(c):
            return partial_ref[pl.ds(c * m_out, m_out), :]

        comm_ref[0] = chunk(lax.rem(mz + NZ - 1, NZ)).astype(jnp.bfloat16)
        for h in range(NZ - 1):
            s_slot, r_slot = h % 2, (h + 1) % 2
            if h > 0:
                pl.semaphore_wait(credit1, 1)
            rdma = pltpu.make_async_remote_copy(
                src_ref=comm_ref.at[s_slot],
                dst_ref=comm_ref.at[r_slot],
                send_sem=send1.at[h],
                recv_sem=recv1.at[h],
                device_id=(mx, my, zr),
                device_id_type=MESH,
            )
            rdma.start()
            rdma.wait()
            if h < NZ - 2:
                pl.semaphore_signal(credit1, inc=1, device_id=(mx, my, zl),
                                    device_id_type=MESH)
            comm_ref[r_slot] = (
                comm_ref[r_slot].astype(jnp.float32)
                + chunk(lax.rem(mz + 2 * NZ - 2 - h, NZ))
            ).astype(jnp.bfloat16)

        commA[1] = comm_ref[1, :, :n_half]
        commB[1] = comm_ref[1, :, n_half:]

        pl.semaphore_signal(creditA, inc=1, device_id=(*prv, mz),
                            device_id_type=MESH)
        pl.semaphore_signal(creditB, inc=1, device_id=(*nxt, mz),
                            device_id_type=MESH)
        for k in range(NR - 1):
            s_slot, r_slot = (1 + k) % 2, k % 2
            pl.semaphore_wait(creditA, 1)
            pl.semaphore_wait(creditB, 1)
            rdma_a = pltpu.make_async_remote_copy(
                src_ref=commA.at[s_slot],
                dst_ref=commA.at[r_slot],
                send_sem=sendA.at[k],
                recv_sem=recvA.at[k],
                device_id=(*nxt, mz),
                device_id_type=MESH,
            )
            rdma_b = pltpu.make_async_remote_copy(
                src_ref=commB.at[s_slot],
                dst_ref=commB.at[r_slot],
                send_sem=sendB.at[k],
                recv_sem=recvB.at[k],
                device_id=(*prv, mz),
                device_id_type=MESH,
            )
            rdma_a.start()
            rdma_b.start()
            if k == 0:
                out_ref[:, pl.ds(p * n_grp, n_grp)] = comm_ref[1].astype(jnp.float32)
            else:
                oa = lax.rem(p + NR - k, NR)
                ob = lax.rem(p + k, NR)
                out_ref[:, pl.ds(oa * n_grp, n_half)] = commA[s_slot].astype(jnp.float32)
                out_ref[:, pl.ds(ob * n_grp + n_half, n_half)] = commB[s_slot].astype(jnp.float32)
            rdma_a.wait()
            rdma_b.wait()
            if k < NR - 2:
                pl.semaphore_signal(creditA, inc=1, device_id=(*prv, mz),
                                    device_id_type=MESH)
                pl.semaphore_signal(creditB, inc=1, device_id=(*nxt, mz),
                                    device_id_type=MESH)
        oa = lax.rem(p + 1, NR)
        ob = lax.rem(p + NR - 1, NR)
        out_ref[:, pl.ds(oa * n_grp, n_half)] = commA[0].astype(jnp.float32)
        out_ref[:, pl.ds(ob * n_grp + n_half, n_half)] = commB[0].astype(jnp.float32)

    return pl.pallas_call(
        body,
        out_shape=jax.ShapeDtypeStruct((m_out, n_tot), jnp.float32),
        in_specs=[pl.BlockSpec(memory_space=pltpu.MemorySpace.VMEM)],
        out_specs=pl.BlockSpec(memory_space=pltpu.MemorySpace.VMEM),
        scratch_shapes=[
            pltpu.VMEM((2, m_out, n_grp), jnp.bfloat16),
            pltpu.VMEM((2, m_out, n_grp // 2), jnp.bfloat16),
            pltpu.VMEM((2, m_out, n_grp // 2), jnp.bfloat16),
            pltpu.SemaphoreType.DMA((NZ - 1,)),
            pltpu.SemaphoreType.DMA((NZ - 1,)),
            pltpu.SemaphoreType.DMA((NR - 1,)),
            pltpu.SemaphoreType.DMA((NR - 1,)),
            pltpu.SemaphoreType.DMA((NR - 1,)),
            pltpu.SemaphoreType.DMA((NR - 1,)),
            pltpu.SemaphoreType.REGULAR,
            pltpu.SemaphoreType.REGULAR,
            pltpu.SemaphoreType.REGULAR,
        ],
        compiler_params=pltpu.CompilerParams(
            collective_id=0, vmem_limit_bytes=48 * 1024 * 1024
        ),
    )(partial)
